# baseline (speedup 1.0000x reference)
import functools

import jax
import jax.numpy as jnp
from jax import lax
from jax.experimental import pallas as pl
from jax.experimental.pallas import tpu as pltpu
from jax.experimental.pallas import tpu_sc as plsc

N = 200704
NG = N // 4
NC = 2
NS = 16
NW = NC * NS
B_PER_W = N // NW
CHUNK = 128
CHUNKS = B_PER_W // CHUNK
L = 16
GROUPS = CHUNK // L
OUT_ROWS = 2 * B_PER_W // 128

_mesh = plsc.VectorSubcoreMesh(core_axis_name="c", subcore_axis_name="s")


@functools.partial(
    pl.kernel,
    mesh=_mesh,
    compiler_params=pltpu.CompilerParams(
        use_tc_tiling_on_sc=False, needs_layout_passes=False
    ),
    out_type=jax.ShapeDtypeStruct((NW * OUT_ROWS, 128), jnp.float32),
    scratch_types=[
        pltpu.VMEM((CHUNKS, CHUNK), jnp.int32),
        pltpu.VMEM((CHUNKS, CHUNK), jnp.int32),
        pltpu.VMEM((B_PER_W, 8), jnp.float32),
        pltpu.VMEM((OUT_ROWS, 128), jnp.float32),
        pltpu.SemaphoreType.DMA,
    ],
)
def _gather_kernel(table_hbm, idx_hbm, out_hbm, idx_v, g_v, rows_v, out_v, sem):
    wid = lax.axis_index("s") * NC + lax.axis_index("c")
    pltpu.sync_copy(idx_hbm.at[pl.ds(wid * CHUNKS, CHUNKS)], idx_v)

    @pl.loop(0, CHUNKS)
    def _fire(j):
        for k in range(GROUPS):
            v = idx_v[j, pl.ds(L * k, L)]
            g_v[j, pl.ds(L * k, L)] = lax.shift_right_logical(v, 2)
        pltpu.async_copy(
            table_hbm.at[g_v.at[j]], rows_v.at[pl.ds(j * CHUNK, CHUNK)], sem
        )

    pltpu.make_async_copy(table_hbm.at[pl.ds(0, B_PER_W)], rows_v, sem).wait()

    @pl.loop(0, CHUNKS)
    def _extract(j):
        for k in range(GROUPS):
            v = idx_v[j, pl.ds(L * k, L)]
            off2 = lax.shift_left(jnp.bitwise_and(v, 3), 1)
            row = lax.iota(jnp.int32, L) + (j * CHUNK + L * k)
            x = plsc.load_gather(rows_v, [row, off2])
            y = plsc.load_gather(rows_v, [row, off2 + 1])
            px = lax.shift_left(row, 1)
            plsc.store_scatter(
                out_v, [lax.shift_right_logical(px, 7), jnp.bitwise_and(px, 127)], x
            )
            py = px + 1
            plsc.store_scatter(
                out_v, [lax.shift_right_logical(py, 7), jnp.bitwise_and(py, 127)], y
            )

    pltpu.sync_copy(out_v, out_hbm.at[pl.ds(wid * OUT_ROWS, OUT_ROWS)])


def kernel(positions, perm):
    table = positions.reshape(NG, 8)
    idx = perm.astype(jnp.int32).reshape(N // CHUNK, CHUNK)
    return _gather_kernel(table, idx).reshape(N, 2)


# speedup vs baseline: 1.2931x; 1.2931x over previous
import functools

import jax
import jax.numpy as jnp
from jax import lax
from jax.experimental import pallas as pl
from jax.experimental.pallas import tpu as pltpu
from jax.experimental.pallas import tpu_sc as plsc

N = 200704
NG = N // 4
NC = 2
NS = 16
NW = NC * NS
B_PER_W = N // NW
CHUNK = 128
CHUNKS = B_PER_W // CHUNK
L = 16
GROUPS = CHUNK // L
OUT_ROWS = 2 * B_PER_W // 128
INTER_STRIDE = 104
BLOCKS = B_PER_W // 8
BANK = 16

_mesh = plsc.VectorSubcoreMesh(core_axis_name="c", subcore_axis_name="s")


@functools.partial(
    pl.kernel,
    mesh=_mesh,
    compiler_params=pltpu.CompilerParams(
        use_tc_tiling_on_sc=False, needs_layout_passes=False
    ),
    out_type=jax.ShapeDtypeStruct((NW * INTER_STRIDE, 128), jnp.float32),
    scratch_types=[
        pltpu.VMEM((CHUNKS, CHUNK), jnp.int32),
        pltpu.VMEM((CHUNKS, CHUNK), jnp.int32),
        pltpu.VMEM((B_PER_W, 8), jnp.float32),
        pltpu.VMEM((OUT_ROWS, 128), jnp.float32),
        pltpu.SemaphoreType.DMA,
    ],
)
def _gather_kernel(table_hbm, idx_hbm, out_hbm, idx_v, g_v, rows_v, out_v, sem):
    wid = lax.axis_index("s") * NC + lax.axis_index("c")
    pltpu.sync_copy(idx_hbm.at[pl.ds(wid * CHUNKS, CHUNKS)], idx_v)

    @pl.loop(0, CHUNKS)
    def _fire(j):
        for k in range(GROUPS):
            v = idx_v[j, pl.ds(L * k, L)]
            g_v[j, pl.ds(L * k, L)] = lax.shift_right_logical(v, 2)
        pltpu.async_copy(
            table_hbm.at[g_v.at[j]], rows_v.at[pl.ds(j * CHUNK, CHUNK)], sem
        )

    pltpu.make_async_copy(table_hbm.at[pl.ds(0, B_PER_W)], rows_v, sem).wait()

    @pl.loop(0, CHUNKS)
    def _extract(j):
        for k in range(GROUPS):
            v = idx_v[j, pl.ds(L * k, L)]
            off2 = lax.shift_left(jnp.bitwise_and(v, 3), 1)
            row = lax.iota(jnp.int32, L) + (j * CHUNK + L * k)
            x = plsc.load_gather(rows_v, [row, off2])
            y = plsc.load_gather(rows_v, [row, off2 + 1])
            px = lax.shift_left(row, 1)
            plsc.store_scatter(
                out_v, [lax.shift_right_logical(px, 7), jnp.bitwise_and(px, 127)], x
            )
            py = px + 1
            plsc.store_scatter(
                out_v, [lax.shift_right_logical(py, 7), jnp.bitwise_and(py, 127)], y
            )

    pltpu.sync_copy(out_v, out_hbm.at[pl.ds(wid * INTER_STRIDE, OUT_ROWS)])


@functools.partial(
    pl.kernel,
    mesh=_mesh,
    compiler_params=pltpu.CompilerParams(
        use_tc_tiling_on_sc=True, needs_layout_passes=False
    ),
    out_type=jax.ShapeDtypeStruct((N, 2), jnp.float32),
    scratch_types=[
        pltpu.VMEM((INTER_STRIDE, 128), jnp.float32),
        pltpu.VMEM((2 * BANK, 8, 2), jnp.float32),
        pltpu.SemaphoreType.DMA,
    ],
)
def _store_kernel(inter_hbm, out_hbm, a_v, blk_v, sem):
    wid = lax.axis_index("s") * NC + lax.axis_index("c")
    pltpu.sync_copy(
        inter_hbm.at[
            pl.ds(pl.multiple_of(wid * INTER_STRIDE, 8), INTER_STRIDE)
        ],
        a_v,
    )
    out_base = pl.multiple_of(wid * B_PER_W, 8)

    lane = lax.iota(jnp.int32, L)
    rws = lax.shift_right_logical(lane, 1)
    cls = jnp.bitwise_and(lane, 1)

    @pl.loop(0, BLOCKS // BANK)
    def _iter(i):
        bank = jnp.bitwise_and(i, 1) * BANK

        @pl.when(i >= 2)
        def _drain():
            for b in range(BANK):
                q = (i - 2) * BANK + b
                pltpu.make_async_copy(
                    blk_v.at[bank + b],
                    out_hbm.at[pl.ds(pl.multiple_of(out_base + q * 8, 8), 8)],
                    sem,
                ).wait()

        for b in range(BANK):
            q = i * BANK + b
            f = lax.shift_left(q, 4)
            vals = a_v[
                lax.shift_right_logical(f, 7), pl.ds(jnp.bitwise_and(f, 127), L)
            ]
            plsc.store_scatter(blk_v.at[bank + b], [rws, cls], vals)
            pltpu.async_copy(
                blk_v.at[bank + b],
                out_hbm.at[pl.ds(pl.multiple_of(out_base + q * 8, 8), 8)],
                sem,
            )

    @pl.loop(BLOCKS // BANK - 2, BLOCKS // BANK)
    def _tail(i):
        bank = jnp.bitwise_and(i, 1) * BANK
        for b in range(BANK):
            q = i * BANK + b
            pltpu.make_async_copy(
                blk_v.at[bank + b],
                out_hbm.at[pl.ds(pl.multiple_of(out_base + q * 8, 8), 8)],
                sem,
            ).wait()


def kernel(positions, perm):
    table = positions.reshape(NG, 8)
    idx = perm.astype(jnp.int32).reshape(N // CHUNK, CHUNK)
    inter = _gather_kernel(table, idx)
    return _store_kernel(inter)


# speedup vs baseline: 1.2935x; 1.0003x over previous
import functools

import jax
import jax.numpy as jnp
from jax import lax
from jax.experimental import pallas as pl
from jax.experimental.pallas import tpu as pltpu
from jax.experimental.pallas import tpu_sc as plsc

N = 200704
NG = N // 4
NC = 2
NS = 16
NW = NC * NS
B_PER_W = N // NW
CHUNK = 128
CHUNKS = B_PER_W // CHUNK
L = 16
GROUPS = CHUNK // L
OUT_ROWS = 2 * B_PER_W // 128
INTER_STRIDE = 104
BLOCKS = B_PER_W // 8
BANK = 16

_mesh = plsc.VectorSubcoreMesh(core_axis_name="c", subcore_axis_name="s")


@functools.partial(
    pl.kernel,
    mesh=_mesh,
    compiler_params=pltpu.CompilerParams(
        use_tc_tiling_on_sc=False, needs_layout_passes=False
    ),
    out_type=jax.ShapeDtypeStruct((NW * INTER_STRIDE, 128), jnp.float32),
    scratch_types=[
        pltpu.VMEM((CHUNKS, CHUNK), jnp.int32),
        pltpu.VMEM((CHUNKS, CHUNK), jnp.int32),
        pltpu.VMEM((B_PER_W, 8), jnp.float32),
        pltpu.VMEM((OUT_ROWS, 128), jnp.float32),
        pltpu.SemaphoreType.DMA,
    ],
)
def _gather_kernel(table_hbm, idx_hbm, out_hbm, idx_v, g_v, rows_v, out_v, sem):
    wid = lax.axis_index("s") * NC + lax.axis_index("c")
    pltpu.sync_copy(idx_hbm.at[pl.ds(wid * CHUNKS, CHUNKS)], idx_v)

    @pl.loop(0, CHUNKS)
    def _fire(j):
        for k in range(GROUPS):
            v = idx_v[j, pl.ds(L * k, L)]
            g_v[j, pl.ds(L * k, L)] = lax.shift_right_logical(v, 2)
        pltpu.async_copy(
            table_hbm.at[g_v.at[j]], rows_v.at[pl.ds(j * CHUNK, CHUNK)], sem
        )

    pltpu.make_async_copy(table_hbm.at[pl.ds(0, B_PER_W)], rows_v, sem).wait()

    @pl.loop(0, CHUNKS)
    def _extract(j):
        for k in range(GROUPS):
            v = idx_v[j, pl.ds(L * k, L)]
            off2 = lax.shift_left(jnp.bitwise_and(v, 3), 1)
            row = lax.iota(jnp.int32, L) + (j * CHUNK + L * k)
            x = plsc.load_gather(rows_v, [row, off2])
            y = plsc.load_gather(rows_v, [row, off2 + 1])
            px = lax.shift_left(row, 1)
            plsc.store_scatter(
                out_v, [lax.shift_right_logical(px, 7), jnp.bitwise_and(px, 127)], x
            )
            py = px + 1
            plsc.store_scatter(
                out_v, [lax.shift_right_logical(py, 7), jnp.bitwise_and(py, 127)], y
            )

    pltpu.sync_copy(out_v, out_hbm.at[pl.ds(wid * INTER_STRIDE, OUT_ROWS)])


SLAB = 64
SLABS = B_PER_W // SLAB
NBUF = 4


@functools.partial(
    pl.kernel,
    mesh=_mesh,
    compiler_params=pltpu.CompilerParams(
        use_tc_tiling_on_sc=True, needs_layout_passes=False
    ),
    out_type=jax.ShapeDtypeStruct((N, 2), jnp.float32),
    scratch_types=[
        pltpu.VMEM((INTER_STRIDE, 128), jnp.float32),
        pltpu.VMEM((NBUF, SLAB, 2), jnp.float32),
        pltpu.SemaphoreType.DMA,
    ],
)
def _store_kernel(inter_hbm, out_hbm, a_v, slab_v, sem):
    wid = lax.axis_index("s") * NC + lax.axis_index("c")
    pltpu.sync_copy(
        inter_hbm.at[
            pl.ds(pl.multiple_of(wid * INTER_STRIDE, 8), INTER_STRIDE)
        ],
        a_v,
    )
    out_base = pl.multiple_of(wid * B_PER_W, 8)

    lane = lax.iota(jnp.int32, L)

    @pl.loop(0, SLABS)
    def _slab(q):
        b = jnp.bitwise_and(q, NBUF - 1)

        @pl.when(q >= NBUF)
        def _drain():
            pltpu.make_async_copy(
                slab_v.at[b],
                out_hbm.at[
                    pl.ds(pl.multiple_of(out_base + (q - NBUF) * SLAB, 8), SLAB)
                ],
                sem,
            ).wait()

        for m in range(8):
            vals = a_v[q, pl.ds(L * m, L)]
            rws = (8 * m) + lax.shift_right_logical(lane, 1)
            cls = jnp.bitwise_and(lane, 1)
            plsc.store_scatter(slab_v.at[b], [rws, cls], vals)

        pltpu.async_copy(
            slab_v.at[b],
            out_hbm.at[pl.ds(pl.multiple_of(out_base + q * SLAB, 8), SLAB)],
            sem,
        )

    @pl.loop(SLABS - NBUF, SLABS)
    def _tail(q):
        b = jnp.bitwise_and(q, NBUF - 1)
        pltpu.make_async_copy(
            slab_v.at[b],
            out_hbm.at[pl.ds(pl.multiple_of(out_base + q * SLAB, 8), SLAB)],
            sem,
        ).wait()


def kernel(positions, perm):
    table = positions.reshape(NG, 8)
    idx = perm.astype(jnp.int32).reshape(N // CHUNK, CHUNK)
    inter = _gather_kernel(table, idx)
    return _store_kernel(inter)


# speedup vs baseline: 1.2960x; 1.0019x over previous
import functools

import jax
import jax.numpy as jnp
from jax import lax
from jax.experimental import pallas as pl
from jax.experimental.pallas import tpu as pltpu
from jax.experimental.pallas import tpu_sc as plsc

N = 200704
NG = N // 8
NC = 2
NS = 16
NW = NC * NS
B_PER_W = N // NW
CHUNK = 128
CHUNKS = B_PER_W // CHUNK
L = 16
GROUPS = CHUNK // L
OUT_ROWS = 2 * B_PER_W // 128
INTER_STRIDE = 104
BLOCKS = B_PER_W // 8
BANK = 16

_mesh = plsc.VectorSubcoreMesh(core_axis_name="c", subcore_axis_name="s")


@functools.partial(
    pl.kernel,
    mesh=_mesh,
    compiler_params=pltpu.CompilerParams(
        use_tc_tiling_on_sc=False, needs_layout_passes=False
    ),
    out_type=jax.ShapeDtypeStruct((NW * INTER_STRIDE, 128), jnp.float32),
    scratch_types=[
        pltpu.VMEM((CHUNKS, CHUNK), jnp.int32),
        pltpu.VMEM((CHUNKS, CHUNK), jnp.int32),
        pltpu.VMEM((B_PER_W, 16), jnp.float32),
        pltpu.VMEM((OUT_ROWS, 128), jnp.float32),
        pltpu.SemaphoreType.DMA,
    ],
)
def _gather_kernel(table_hbm, idx_hbm, out_hbm, idx_v, g_v, rows_v, out_v, sem):
    wid = lax.axis_index("s") * NC + lax.axis_index("c")
    pltpu.sync_copy(idx_hbm.at[pl.ds(wid * CHUNKS, CHUNKS)], idx_v)

    @pl.loop(0, CHUNKS)
    def _fire(j):
        for k in range(GROUPS):
            v = idx_v[j, pl.ds(L * k, L)]
            g_v[j, pl.ds(L * k, L)] = lax.shift_right_logical(v, 3)
        pltpu.async_copy(
            table_hbm.at[g_v.at[j]], rows_v.at[pl.ds(j * CHUNK, CHUNK)], sem
        )

    pltpu.make_async_copy(table_hbm.at[pl.ds(0, B_PER_W)], rows_v, sem).wait()

    @pl.loop(0, CHUNKS)
    def _extract(j):
        for k in range(GROUPS):
            v = idx_v[j, pl.ds(L * k, L)]
            off2 = lax.shift_left(jnp.bitwise_and(v, 7), 1)
            row = lax.iota(jnp.int32, L) + (j * CHUNK + L * k)
            x = plsc.load_gather(rows_v, [row, off2])
            y = plsc.load_gather(rows_v, [row, off2 + 1])
            px = lax.shift_left(row, 1)
            plsc.store_scatter(
                out_v, [lax.shift_right_logical(px, 7), jnp.bitwise_and(px, 127)], x
            )
            py = px + 1
            plsc.store_scatter(
                out_v, [lax.shift_right_logical(py, 7), jnp.bitwise_and(py, 127)], y
            )

    pltpu.sync_copy(out_v, out_hbm.at[pl.ds(wid * INTER_STRIDE, OUT_ROWS)])


SLAB = 64
SLABS = B_PER_W // SLAB
NBUF = 4


@functools.partial(
    pl.kernel,
    mesh=_mesh,
    compiler_params=pltpu.CompilerParams(
        use_tc_tiling_on_sc=True, needs_layout_passes=False
    ),
    out_type=jax.ShapeDtypeStruct((N, 2), jnp.float32),
    scratch_types=[
        pltpu.VMEM((INTER_STRIDE, 128), jnp.float32),
        pltpu.VMEM((NBUF, SLAB, 2), jnp.float32),
        pltpu.SemaphoreType.DMA,
    ],
)
def _store_kernel(inter_hbm, out_hbm, a_v, slab_v, sem):
    wid = lax.axis_index("s") * NC + lax.axis_index("c")
    pltpu.sync_copy(
        inter_hbm.at[
            pl.ds(pl.multiple_of(wid * INTER_STRIDE, 8), INTER_STRIDE)
        ],
        a_v,
    )
    out_base = pl.multiple_of(wid * B_PER_W, 8)

    lane = lax.iota(jnp.int32, L)

    @pl.loop(0, SLABS)
    def _slab(q):
        b = jnp.bitwise_and(q, NBUF - 1)

        @pl.when(q >= NBUF)
        def _drain():
            pltpu.make_async_copy(
                slab_v.at[b],
                out_hbm.at[
                    pl.ds(pl.multiple_of(out_base + (q - NBUF) * SLAB, 8), SLAB)
                ],
                sem,
            ).wait()

        for m in range(8):
            vals = a_v[q, pl.ds(L * m, L)]
            rws = (8 * m) + lax.shift_right_logical(lane, 1)
            cls = jnp.bitwise_and(lane, 1)
            plsc.store_scatter(slab_v.at[b], [rws, cls], vals)

        pltpu.async_copy(
            slab_v.at[b],
            out_hbm.at[pl.ds(pl.multiple_of(out_base + q * SLAB, 8), SLAB)],
            sem,
        )

    @pl.loop(SLABS - NBUF, SLABS)
    def _tail(q):
        b = jnp.bitwise_and(q, NBUF - 1)
        pltpu.make_async_copy(
            slab_v.at[b],
            out_hbm.at[pl.ds(pl.multiple_of(out_base + q * SLAB, 8), SLAB)],
            sem,
        ).wait()


def kernel(positions, perm):
    table = positions.reshape(NG, 16)
    idx = perm.astype(jnp.int32).reshape(N // CHUNK, CHUNK)
    inter = _gather_kernel(table, idx)
    return _store_kernel(inter)
